# Initial kernel scaffold; baseline (speedup 1.0000x reference)
import jax
import jax.numpy as jnp
from jax.experimental import pallas as pl


def kernel(x, edge_index, batch, params):
    raise NotImplementedError("write your pallas kernel here")


# speedup vs baseline: 3.7119x; 3.7119x over previous
import functools

import jax
import jax.numpy as jnp
from jax import lax
from jax.experimental import pallas as pl
from jax.experimental.pallas import tpu as pltpu
from jax.experimental.pallas import tpu_sc as plsc

N = 10000
N_PAD = 10240
E = 320000
NCORES = 2
NSUB = 16
NW = NCORES * NSUB
EPW = E // NW
CHUNK = 80
NCHUNK = EPW // CHUNK
ROWS_PER_TILE = N_PAD // NSUB
SOW = 160


def _make_segsum(W):
    mesh = plsc.VectorSubcoreMesh(core_axis_name="c", subcore_axis_name="s",
                                  num_cores=NCORES, num_subcores=NSUB)

    @functools.partial(
        pl.kernel,
        out_type=jax.ShapeDtypeStruct((NCORES, N_PAD, W), jnp.float32),
        mesh=mesh,
        compiler_params=pltpu.CompilerParams(use_tc_tiling_on_sc=False),
        scratch_types=[
            pltpu.VMEM((CHUNK,), jnp.int32),
            pltpu.VMEM((CHUNK,), jnp.int32),
            pltpu.VMEM((CHUNK, W), jnp.float32),
            pltpu.VMEM_SHARED((N_PAD, W), jnp.float32),
            pltpu.SemaphoreType.DMA,
        ],
    )
    def seg(vals, src, dst, zeros, out, idx_s, idx_d, rows, acc, sem):
        cid = lax.axis_index("c")
        sid = lax.axis_index("s")
        wid = cid * NSUB + sid
        r0 = sid * ROWS_PER_TILE
        pltpu.sync_copy(zeros.at[pl.ds(r0, ROWS_PER_TILE)],
                        acc.at[pl.ds(r0, ROWS_PER_TILE)])
        plsc.subcore_barrier()

        base = wid * EPW

        def step(i, carry):
            off = pl.multiple_of(base + i * CHUNK, 8)
            pltpu.sync_copy(src.at[pl.ds(off, CHUNK)], idx_s)
            pltpu.sync_copy(dst.at[pl.ds(off, CHUNK)], idx_d)
            pltpu.async_copy(vals.at[idx_s], rows, sem).wait()
            pltpu.sync_copy(rows, acc.at[idx_d], add=True)
            return carry

        lax.fori_loop(0, NCHUNK, step, 0)
        plsc.subcore_barrier()
        pltpu.sync_copy(acc.at[pl.ds(r0, ROWS_PER_TILE)],
                        out.at[cid, pl.ds(r0, ROWS_PER_TILE)])

    return seg


def _make_segsum_split(W):
    EPT = E // NSUB
    NCH = EPT // CHUNK
    mesh = plsc.VectorSubcoreMesh(core_axis_name="c", subcore_axis_name="s",
                                  num_cores=NCORES, num_subcores=NSUB)

    @functools.partial(
        pl.kernel,
        out_type=jax.ShapeDtypeStruct((NCORES, N_PAD, W), jnp.float32),
        mesh=mesh,
        compiler_params=pltpu.CompilerParams(use_tc_tiling_on_sc=False),
        scratch_types=[
            pltpu.VMEM((CHUNK,), jnp.int32),
            pltpu.VMEM((CHUNK,), jnp.int32),
            pltpu.VMEM((CHUNK, W), jnp.float32),
            pltpu.VMEM_SHARED((N_PAD, W), jnp.float32),
            pltpu.SemaphoreType.DMA,
        ],
    )
    def seg(vals, src, dst, zeros, out, idx_s, idx_d, rows, acc, sem):
        cid = lax.axis_index("c")
        sid = lax.axis_index("s")
        r0 = sid * ROWS_PER_TILE
        pltpu.sync_copy(zeros.at[pl.ds(r0, ROWS_PER_TILE)],
                        acc.at[pl.ds(r0, ROWS_PER_TILE)])
        plsc.subcore_barrier()

        base = sid * EPT
        bias = cid * N

        def step(i, carry):
            off = pl.multiple_of(base + i * CHUNK, 8)
            pltpu.sync_copy(src.at[pl.ds(off, CHUNK)], idx_s)
            pltpu.sync_copy(dst.at[pl.ds(off, CHUNK)], idx_d)
            for j in range(CHUNK // 16):
                sl = pl.ds(j * 16, 16)
                idx_s[sl] = idx_s[sl] + bias
            pltpu.async_copy(vals.at[idx_s], rows, sem).wait()
            pltpu.sync_copy(rows, acc.at[idx_d], add=True)
            return carry

        lax.fori_loop(0, NCH, step, 0)
        plsc.subcore_barrier()
        pltpu.sync_copy(acc.at[pl.ds(r0, ROWS_PER_TILE)],
                        out.at[cid, pl.ds(r0, ROWS_PER_TILE)])

    return seg


_SEGSUM_CACHE = {}


def _segsum(vals, src, dst):
    W = vals.shape[1]
    if W not in _SEGSUM_CACHE:
        _SEGSUM_CACHE[W] = _make_segsum(W)
    zeros = jnp.zeros((N_PAD, W), jnp.float32)
    return _SEGSUM_CACHE[W](vals, src, dst, zeros)


def _segsum_oc(vals2n, src, dst):
    if 'oc' not in _SEGSUM_CACHE:
        _SEGSUM_CACHE['oc'] = _make_segsum_split(SOW)
    zeros = jnp.zeros((N_PAD, SOW), jnp.float32)
    return _SEGSUM_CACHE['oc'](vals2n, src, dst, zeros)


def _leaky(t):
    return jnp.where(t >= 0, t, 0.01 * t)


def _bn(t, g, b):
    mu = jnp.mean(t, axis=0, keepdims=True)
    var = jnp.mean((t - mu) ** 2, axis=0, keepdims=True)
    return (t - mu) / jnp.sqrt(var + 1e-5) * g + b


def _lin_body(aggp, x1, wrel, wroot, b, o):
    agg = aggp[0] + aggp[1]
    h = (jnp.dot(agg, wrel[...], preferred_element_type=jnp.float32)
         + jnp.dot(x1[...], wroot[...], preferred_element_type=jnp.float32)
         + b[...])
    o[...] = _leaky(h)


def _bn_body(t, g, be, o):
    o[...] = _bn(t[...], g[...], be[...])


def _som_body(x1, x2, x3, s1t, s1n, s2t, s2n, s3t, s3n, o_so):
    def dists(xi, wt, wn):
        d2 = (jnp.sum(xi * xi, axis=1, keepdims=True)
              - 2.0 * jnp.dot(xi, wt, preferred_element_type=jnp.float32)
              + wn)
        return jnp.sqrt(jnp.maximum(d2, 1e-12))

    so1 = dists(x1[...], s1t[...], s1n[...])[:, :100]
    so2 = dists(x2[...], s2t[...], s2n[...])[:, :100]
    so3 = dists(x3[...], s3t[...], s3n[...])[:, :100]
    zpad = jnp.zeros((so1.shape[0], 20), jnp.float32)
    o_so[0, :, :] = jnp.concatenate([so1, so2[:, :60]], axis=1)
    o_so[1, :, :] = jnp.concatenate([so2[:, 60:], so3, zpad], axis=1)


def _oc_body(aggso, sost, wr1, wo1, wr2, wo2, wr3, wo3, b, o):
    a0, a1 = aggso[0], aggso[1]
    s0, s1 = sost[0], sost[1]
    agg1, so1 = a0[:, :100], s0[:, :100]
    agg2 = jnp.concatenate([a0[:, 100:160], a1[:, :40]], axis=1)
    so2 = jnp.concatenate([s0[:, 100:160], s1[:, :40]], axis=1)
    agg3, so3 = a1[:, 40:140], s1[:, 40:140]

    def part(agg, so, wr, wo):
        return (jnp.dot(agg, wr[...], preferred_element_type=jnp.float32)
                + jnp.dot(so, wo[...], preferred_element_type=jnp.float32))

    t = jnp.concatenate([part(agg1, so1, wr1, wo1),
                         part(agg2, so2, wr2, wo2),
                         part(agg3, so3, wr3, wo3)], axis=1) + b[...]
    o[...] = _leaky(t)


PRB = 2000
PNSTEP = N // PRB


def _pool_body(hc_ref, som_ref, batch_ref, wgnn, bgnn, wout, bout,
               o_h, o_gnn, s_conv, s_som, mx_c, mx_s, cnt):
    i = pl.program_id(0)

    @pl.when(i == 0)
    def _init():
        s_conv[...] = jnp.zeros((64, 384), jnp.float32)
        s_som[...] = jnp.zeros((64, 192), jnp.float32)
        mx_c[...] = jnp.full((64, 384), -jnp.inf, jnp.float32)
        mx_s[...] = jnp.full((64, 192), -jnp.inf, jnp.float32)
        cnt[...] = jnp.zeros((64, 1), jnp.float32)

    bvec = batch_ref[...]
    hc = hc_ref[...]
    som = som_ref[...]
    oh = (bvec == lax.broadcasted_iota(jnp.int32, (PRB, 64), 1)
          ).astype(jnp.float32)

    def segsum(hm):
        return lax.dot_general(oh, hm, (((0,), (0,)), ((), ())),
                               preferred_element_type=jnp.float32,
                               precision=lax.Precision.HIGHEST)

    s_conv[...] += segsum(hc)
    s_som[...] += segsum(som)
    cnt[...] += lax.dot_general(oh, jnp.ones((PRB, 1), jnp.float32),
                                (((0,), (0,)), ((), ())),
                                preferred_element_type=jnp.float32,
                                precision=lax.Precision.HIGHEST)
    row_iota = lax.broadcasted_iota(jnp.int32, (64, 1), 0)

    def mxstep(gidx, carry):
        mc = jnp.max(jnp.where(bvec == gidx, hc, -jnp.inf),
                     axis=0, keepdims=True)
        ms = jnp.max(jnp.where(bvec == gidx, som, -jnp.inf),
                     axis=0, keepdims=True)
        sel = row_iota == gidx
        mx_c[...] = jnp.where(sel, jnp.maximum(mx_c[...], mc), mx_c[...])
        mx_s[...] = jnp.where(sel, jnp.maximum(mx_s[...], ms), mx_s[...])
        return carry

    lax.fori_loop(0, 64, mxstep, 0)

    @pl.when(i == PNSTEP - 1)
    def _heads():
        cn = jnp.maximum(cnt[...], 1.0)
        sc = s_conv[...]
        ss = s_som[...]
        h_gnn = jnp.concatenate([sc / cn, sc, mx_c[...]], axis=1)
        hp = jnp.concatenate([ss / cn, ss, mx_s[...]], axis=1)

        def logsoftmax(z):
            zm = jnp.max(z, axis=1, keepdims=True)
            return (z - zm) - jnp.log(jnp.sum(jnp.exp(z - zm), axis=1,
                                              keepdims=True))

        o_gnn[...] = logsoftmax(
            jnp.dot(h_gnn, wgnn[...], preferred_element_type=jnp.float32)
            + bgnn[...])
        o_h[...] = logsoftmax(
            jnp.dot(hp, wout[...], preferred_element_type=jnp.float32)
            + bout[...])


def _tc(body, out_shape, *args):
    return pl.pallas_call(body, out_shape=out_shape)(*args)


def kernel(x, edge_index, batch, params):
    p = params
    src = edge_index[0]
    dst = edge_index[1]

    w1rel, w1root = p['conv1_Wrel'].T, p['conv1_Wroot'].T
    b1 = p['conv1_b'].reshape(1, -1)
    g1, be1 = p['norm1_g'].reshape(1, -1), p['norm1_b'].reshape(1, -1)
    w2rel, w2root = p['conv2_Wrel'].T, p['conv2_Wroot'].T
    b2 = p['conv2_b'].reshape(1, -1)
    g2, be2 = p['norm2_g'].reshape(1, -1), p['norm2_b'].reshape(1, -1)
    w3rel, w3root = p['conv3_Wrel'].T, p['conv3_Wroot'].T
    b3 = p['conv3_b'].reshape(1, -1)
    g3, be3 = p['norm3_g'].reshape(1, -1), p['norm3_b'].reshape(1, -1)

    def som_prep(w):
        wt = jnp.pad(w, ((0, 28), (0, 0))).T
        wn = jnp.pad(jnp.sum(w * w, axis=1), (0, 28)).reshape(1, -1)
        return wt, wn

    s1t, s1n = som_prep(p['som1_W'])
    s2t, s2n = som_prep(p['som2_W'])
    s3t, s3n = som_prep(p['som3_W'])

    wr1, wo1 = p['oc1_Wrel'].T, p['oc1_Wroot'].T
    wr2, wo2 = p['oc2_Wrel'].T, p['oc2_Wroot'].T
    wr3, wo3 = p['oc3_Wrel'].T, p['oc3_Wroot'].T
    boc = jnp.concatenate([p['oc1_b'], p['oc2_b'], p['oc3_b']]).reshape(1, -1)
    goc = jnp.concatenate([p['on1_g'], p['on2_g'], p['on3_g']]).reshape(1, -1)
    beoc = jnp.concatenate([p['on1_b'], p['on2_b'], p['on3_b']]).reshape(1, -1)
    wgnn = p['lin_GNN_W'].T
    bgnn = p['lin_GNN_b'].reshape(1, -1)
    wout = p['lin_out_W'].T
    bout = p['lin_out_b'].reshape(1, -1)
    batch2d = batch.reshape(N, 1)

    f32 = jnp.float32
    RB = 2000
    full = lambda shp: pl.BlockSpec(shp, lambda i: (0,) * len(shp))

    def _lin(aggp, xin, wrel, wroot, bb):
        wa, wx, wo = aggp.shape[2], xin.shape[1], wrel.shape[1]
        return pl.pallas_call(
            _lin_body,
            grid=(N // RB,),
            in_specs=[
                pl.BlockSpec((2, RB, wa), lambda i: (0, i, 0)),
                pl.BlockSpec((RB, wx), lambda i: (i, 0)),
                full((wa, wo)), full((wx, wo)), full((1, wo)),
            ],
            out_specs=pl.BlockSpec((RB, wo), lambda i: (i, 0)),
            out_shape=jax.ShapeDtypeStruct((N, wo), f32),
        )(aggp, xin, wrel, wroot, bb)

    agg1 = _segsum(x, src, dst)
    t1 = _lin(agg1, x, w1rel, w1root, b1)
    x1 = _tc(_bn_body, jax.ShapeDtypeStruct((N, 64), f32), t1, g1, be1)

    agg2 = _segsum(x1, src, dst)
    t2 = _lin(agg2, x1, w2rel, w2root, b2)
    x2 = _tc(_bn_body, jax.ShapeDtypeStruct((N, 128), f32), t2, g2, be2)

    agg3 = _segsum(x2, src, dst)
    t3 = _lin(agg3, x2, w3rel, w3root, b3)
    x3 = _tc(_bn_body, jax.ShapeDtypeStruct((N, 192), f32), t3, g3, be3)

    so_stack = pl.pallas_call(
        _som_body,
        grid=(N // RB,),
        in_specs=[
            pl.BlockSpec((RB, 64), lambda i: (i, 0)),
            pl.BlockSpec((RB, 128), lambda i: (i, 0)),
            pl.BlockSpec((RB, 192), lambda i: (i, 0)),
            full((64, 128)), full((1, 128)),
            full((128, 128)), full((1, 128)),
            full((192, 128)), full((1, 128)),
        ],
        out_specs=pl.BlockSpec((2, RB, SOW), lambda i: (0, i, 0)),
        out_shape=jax.ShapeDtypeStruct((2, N, SOW), f32),
    )(x1, x2, x3, s1t, s1n, s2t, s2n, s3t, s3n)

    agg_so = _segsum_oc(so_stack.reshape(2 * N, SOW), src, dst)

    t_oc = pl.pallas_call(
        _oc_body,
        grid=(N // RB,),
        in_specs=[
            pl.BlockSpec((2, RB, SOW), lambda i: (0, i, 0)),
            pl.BlockSpec((2, RB, SOW), lambda i: (0, i, 0)),
            full((100, 64)), full((100, 64)),
            full((100, 64)), full((100, 64)),
            full((100, 64)), full((100, 64)),
            full((1, 192)),
        ],
        out_specs=pl.BlockSpec((RB, 192), lambda i: (i, 0)),
        out_shape=jax.ShapeDtypeStruct((N, 192), f32),
    )(agg_so, so_stack, wr1, wo1, wr2, wo2, wr3, wo3, boc)
    som_out = _tc(_bn_body, jax.ShapeDtypeStruct((N, 192), f32),
                  t_oc, goc, beoc)

    h_conv = jnp.concatenate([x1, x2, x3], axis=1)
    h, gnn_out = pl.pallas_call(
        _pool_body,
        grid=(PNSTEP,),
        in_specs=[
            pl.BlockSpec((PRB, 384), lambda i: (i, 0)),
            pl.BlockSpec((PRB, 192), lambda i: (i, 0)),
            pl.BlockSpec((PRB, 1), lambda i: (i, 0)),
            full((1152, 2)), full((1, 2)), full((576, 2)), full((1, 2)),
        ],
        out_specs=[full((64, 2)), full((64, 2))],
        out_shape=[jax.ShapeDtypeStruct((64, 2), f32),
                   jax.ShapeDtypeStruct((64, 2), f32)],
        scratch_shapes=[
            pltpu.VMEM((64, 384), f32),
            pltpu.VMEM((64, 192), f32),
            pltpu.VMEM((64, 384), f32),
            pltpu.VMEM((64, 192), f32),
            pltpu.VMEM((64, 1), f32),
        ],
    )(h_conv, som_out, batch2d, wgnn, bgnn, wout, bout)
    return (h, h_conv, gnn_out)


# speedup vs baseline: 5.5920x; 1.5065x over previous
import functools

import jax
import jax.numpy as jnp
from jax import lax
from jax.experimental import pallas as pl
from jax.experimental.pallas import tpu as pltpu
from jax.experimental.pallas import tpu_sc as plsc

N = 10000
N_PAD = 10240
E = 320000
NCORES = 2
NSUB = 16
NW = NCORES * NSUB
EPW = E // NW
CHUNK = 80
NCHUNK = EPW // CHUNK
ROWS_PER_TILE = N_PAD // NSUB
SOW = 160


def _seg_pipeline(vals, acc, src2d, dst2d, rowbase, bias,
                  idx_s, idx_d, rows, gsem, ssem, nch):
    nbuf = len(rows)
    nfull = nch // nbuf
    rem = nch % nbuf

    def stage(j, b):
        pltpu.sync_copy(src2d.at[rowbase + j], idx_s[b])
        pltpu.sync_copy(dst2d.at[rowbase + j], idx_d[b])
        if bias is not None:
            for k in range(CHUNK // 16):
                sl = pl.ds(k * 16, 16)
                idx_s[b][sl] = idx_s[b][sl] + bias
        return pltpu.async_copy(vals.at[idx_s[b]], rows[b], gsem[b])

    def group(i, carry):
        gds = []
        for b in range(nbuf):
            @pl.when(i > 0)
            def _(b=b):
                pltpu.make_async_copy(rows[b], acc.at[idx_d[b]],
                                      ssem[b]).wait()
            gds.append(stage(i * nbuf + b, b))
        for b in range(nbuf):
            gds[b].wait()
            pltpu.async_copy(rows[b], acc.at[idx_d[b]], ssem[b], add=True)
        return carry

    lax.fori_loop(0, nfull, group, 0)
    for b in range(rem):
        if nfull > 0:
            pltpu.make_async_copy(rows[b], acc.at[idx_d[b]], ssem[b]).wait()
        stage(nfull * nbuf + b, b).wait()
        pltpu.async_copy(rows[b], acc.at[idx_d[b]], ssem[b], add=True)
    for b in range(nbuf):
        if nfull > 0 or b < rem:
            pltpu.make_async_copy(rows[b], acc.at[idx_d[b]], ssem[b]).wait()


def _make_segsum_impl(W, split, nbuf):
    NCH = NCHUNK if not split else (E // NSUB) // CHUNK
    mesh = plsc.VectorSubcoreMesh(core_axis_name="c", subcore_axis_name="s",
                                  num_cores=NCORES, num_subcores=NSUB)

    @functools.partial(
        pl.kernel,
        out_type=jax.ShapeDtypeStruct((NCORES, N_PAD, W), jnp.float32),
        mesh=mesh,
        compiler_params=pltpu.CompilerParams(use_tc_tiling_on_sc=False),
        scratch_types=[
            [pltpu.VMEM((CHUNK,), jnp.int32)] * nbuf,
            [pltpu.VMEM((CHUNK,), jnp.int32)] * nbuf,
            [pltpu.VMEM((CHUNK, W), jnp.float32)] * nbuf,
            pltpu.VMEM_SHARED((N_PAD, W), jnp.float32),
            [pltpu.SemaphoreType.DMA] * nbuf,
            [pltpu.SemaphoreType.DMA] * nbuf,
        ],
    )
    def seg(vals, src2d, dst2d, zeros, out,
            idx_s, idx_d, rows, acc, gsem, ssem):
        cid = lax.axis_index("c")
        sid = lax.axis_index("s")
        r0 = sid * ROWS_PER_TILE
        pltpu.sync_copy(zeros.at[pl.ds(r0, ROWS_PER_TILE)],
                        acc.at[pl.ds(r0, ROWS_PER_TILE)])
        plsc.subcore_barrier()
        if split:
            rowbase = sid * NCH
            bias = cid * N
        else:
            rowbase = (cid * NSUB + sid) * NCH
            bias = None
        _seg_pipeline(vals, acc, src2d, dst2d, rowbase, bias,
                      idx_s, idx_d, rows, gsem, ssem, NCH)
        plsc.subcore_barrier()
        pltpu.sync_copy(acc.at[pl.ds(r0, ROWS_PER_TILE)],
                        out.at[cid, pl.ds(r0, ROWS_PER_TILE)])

    return seg


_SEGSUM_CACHE = {}


def _segsum(vals, src2d, dst2d):
    W = vals.shape[1]
    if W not in _SEGSUM_CACHE:
        _SEGSUM_CACHE[W] = _make_segsum_impl(W, split=False, nbuf=4)
    zeros = jnp.zeros((N_PAD, W), jnp.float32)
    return _SEGSUM_CACHE[W](vals, src2d, dst2d, zeros)


def _segsum_oc(vals2n, src2d, dst2d):
    if 'oc' not in _SEGSUM_CACHE:
        _SEGSUM_CACHE['oc'] = _make_segsum_impl(SOW, split=True, nbuf=2)
    zeros = jnp.zeros((N_PAD, SOW), jnp.float32)
    return _SEGSUM_CACHE['oc'](vals2n, src2d, dst2d, zeros)


def _leaky(t):
    return jnp.where(t >= 0, t, 0.01 * t)


def _bn(t, g, b):
    mu = jnp.mean(t, axis=0, keepdims=True)
    var = jnp.mean((t - mu) ** 2, axis=0, keepdims=True)
    return (t - mu) / jnp.sqrt(var + 1e-5) * g + b


def _lin_body(aggp, x1, wrel, wroot, b, o):
    agg = aggp[0] + aggp[1]
    h = (jnp.dot(agg, wrel[...], preferred_element_type=jnp.float32)
         + jnp.dot(x1[...], wroot[...], preferred_element_type=jnp.float32)
         + b[...])
    o[...] = _leaky(h)


def _bn_body(t, g, be, o):
    o[...] = _bn(t[...], g[...], be[...])


def _som_body(x1, x2, x3, s1t, s1n, s2t, s2n, s3t, s3n, o_so):
    def dists(xi, wt, wn):
        d2 = (jnp.sum(xi * xi, axis=1, keepdims=True)
              - 2.0 * jnp.dot(xi, wt, preferred_element_type=jnp.float32)
              + wn)
        return jnp.sqrt(jnp.maximum(d2, 1e-12))

    so1 = dists(x1[...], s1t[...], s1n[...])[:, :100]
    so2 = dists(x2[...], s2t[...], s2n[...])[:, :100]
    so3 = dists(x3[...], s3t[...], s3n[...])[:, :100]
    zpad = jnp.zeros((so1.shape[0], 20), jnp.float32)
    o_so[0, :, :] = jnp.concatenate([so1, so2[:, :60]], axis=1)
    o_so[1, :, :] = jnp.concatenate([so2[:, 60:], so3, zpad], axis=1)


def _oc_body(aggso, sost, wr1, wo1, wr2, wo2, wr3, wo3, b, o):
    a0, a1 = aggso[0], aggso[1]
    s0, s1 = sost[0], sost[1]
    agg1, so1 = a0[:, :100], s0[:, :100]
    agg2 = jnp.concatenate([a0[:, 100:160], a1[:, :40]], axis=1)
    so2 = jnp.concatenate([s0[:, 100:160], s1[:, :40]], axis=1)
    agg3, so3 = a1[:, 40:140], s1[:, 40:140]

    def part(agg, so, wr, wo):
        return (jnp.dot(agg, wr[...], preferred_element_type=jnp.float32)
                + jnp.dot(so, wo[...], preferred_element_type=jnp.float32))

    t = jnp.concatenate([part(agg1, so1, wr1, wo1),
                         part(agg2, so2, wr2, wo2),
                         part(agg3, so3, wr3, wo3)], axis=1) + b[...]
    o[...] = _leaky(t)


PRB = 2000
PNSTEP = N // PRB


def _pool_body(hc_ref, som_ref, batch_ref, wgnn, bgnn, wout, bout,
               o_h, o_gnn, s_conv, s_som, mx_c, mx_s, cnt):
    i = pl.program_id(0)

    @pl.when(i == 0)
    def _init():
        s_conv[...] = jnp.zeros((64, 384), jnp.float32)
        s_som[...] = jnp.zeros((64, 192), jnp.float32)
        mx_c[...] = jnp.full((64, 384), -jnp.inf, jnp.float32)
        mx_s[...] = jnp.full((64, 192), -jnp.inf, jnp.float32)
        cnt[...] = jnp.zeros((64, 1), jnp.float32)

    bvec = batch_ref[...]
    hc = hc_ref[...]
    som = som_ref[...]
    oh = (bvec == lax.broadcasted_iota(jnp.int32, (PRB, 64), 1)
          ).astype(jnp.float32)

    def segsum(hm):
        return lax.dot_general(oh, hm, (((0,), (0,)), ((), ())),
                               preferred_element_type=jnp.float32,
                               precision=lax.Precision.HIGHEST)

    s_conv[...] += segsum(hc)
    s_som[...] += segsum(som)
    cnt[...] += lax.dot_general(oh, jnp.ones((PRB, 1), jnp.float32),
                                (((0,), (0,)), ((), ())),
                                preferred_element_type=jnp.float32,
                                precision=lax.Precision.HIGHEST)
    row_iota = lax.broadcasted_iota(jnp.int32, (64, 1), 0)

    def mxstep(gidx, carry):
        mc = jnp.max(jnp.where(bvec == gidx, hc, -jnp.inf),
                     axis=0, keepdims=True)
        ms = jnp.max(jnp.where(bvec == gidx, som, -jnp.inf),
                     axis=0, keepdims=True)
        sel = row_iota == gidx
        mx_c[...] = jnp.where(sel, jnp.maximum(mx_c[...], mc), mx_c[...])
        mx_s[...] = jnp.where(sel, jnp.maximum(mx_s[...], ms), mx_s[...])
        return carry

    lax.fori_loop(0, 64, mxstep, 0)

    @pl.when(i == PNSTEP - 1)
    def _heads():
        cn = jnp.maximum(cnt[...], 1.0)
        sc = s_conv[...]
        ss = s_som[...]
        h_gnn = jnp.concatenate([sc / cn, sc, mx_c[...]], axis=1)
        hp = jnp.concatenate([ss / cn, ss, mx_s[...]], axis=1)

        def logsoftmax(z):
            zm = jnp.max(z, axis=1, keepdims=True)
            return (z - zm) - jnp.log(jnp.sum(jnp.exp(z - zm), axis=1,
                                              keepdims=True))

        o_gnn[...] = logsoftmax(
            jnp.dot(h_gnn, wgnn[...], preferred_element_type=jnp.float32)
            + bgnn[...])
        o_h[...] = logsoftmax(
            jnp.dot(hp, wout[...], preferred_element_type=jnp.float32)
            + bout[...])


def _tc(body, out_shape, *args):
    return pl.pallas_call(body, out_shape=out_shape)(*args)


def kernel(x, edge_index, batch, params):
    p = params
    src = edge_index[0]
    dst = edge_index[1]
    src2d = src.reshape(E // CHUNK, CHUNK)
    dst2d = dst.reshape(E // CHUNK, CHUNK)

    w1rel, w1root = p['conv1_Wrel'].T, p['conv1_Wroot'].T
    b1 = p['conv1_b'].reshape(1, -1)
    g1, be1 = p['norm1_g'].reshape(1, -1), p['norm1_b'].reshape(1, -1)
    w2rel, w2root = p['conv2_Wrel'].T, p['conv2_Wroot'].T
    b2 = p['conv2_b'].reshape(1, -1)
    g2, be2 = p['norm2_g'].reshape(1, -1), p['norm2_b'].reshape(1, -1)
    w3rel, w3root = p['conv3_Wrel'].T, p['conv3_Wroot'].T
    b3 = p['conv3_b'].reshape(1, -1)
    g3, be3 = p['norm3_g'].reshape(1, -1), p['norm3_b'].reshape(1, -1)

    def som_prep(w):
        wt = jnp.pad(w, ((0, 28), (0, 0))).T
        wn = jnp.pad(jnp.sum(w * w, axis=1), (0, 28)).reshape(1, -1)
        return wt, wn

    s1t, s1n = som_prep(p['som1_W'])
    s2t, s2n = som_prep(p['som2_W'])
    s3t, s3n = som_prep(p['som3_W'])

    wr1, wo1 = p['oc1_Wrel'].T, p['oc1_Wroot'].T
    wr2, wo2 = p['oc2_Wrel'].T, p['oc2_Wroot'].T
    wr3, wo3 = p['oc3_Wrel'].T, p['oc3_Wroot'].T
    boc = jnp.concatenate([p['oc1_b'], p['oc2_b'], p['oc3_b']]).reshape(1, -1)
    goc = jnp.concatenate([p['on1_g'], p['on2_g'], p['on3_g']]).reshape(1, -1)
    beoc = jnp.concatenate([p['on1_b'], p['on2_b'], p['on3_b']]).reshape(1, -1)
    wgnn = p['lin_GNN_W'].T
    bgnn = p['lin_GNN_b'].reshape(1, -1)
    wout = p['lin_out_W'].T
    bout = p['lin_out_b'].reshape(1, -1)
    batch2d = batch.reshape(N, 1)

    f32 = jnp.float32
    RB = 2000
    full = lambda shp: pl.BlockSpec(shp, lambda i: (0,) * len(shp))

    def _lin(aggp, xin, wrel, wroot, bb):
        wa, wx, wo = aggp.shape[2], xin.shape[1], wrel.shape[1]
        return pl.pallas_call(
            _lin_body,
            grid=(N // RB,),
            in_specs=[
                pl.BlockSpec((2, RB, wa), lambda i: (0, i, 0)),
                pl.BlockSpec((RB, wx), lambda i: (i, 0)),
                full((wa, wo)), full((wx, wo)), full((1, wo)),
            ],
            out_specs=pl.BlockSpec((RB, wo), lambda i: (i, 0)),
            out_shape=jax.ShapeDtypeStruct((N, wo), f32),
        )(aggp, xin, wrel, wroot, bb)

    agg1 = _segsum(x, src2d, dst2d)
    t1 = _lin(agg1, x, w1rel, w1root, b1)
    x1 = _tc(_bn_body, jax.ShapeDtypeStruct((N, 64), f32), t1, g1, be1)

    agg2 = _segsum(x1, src2d, dst2d)
    t2 = _lin(agg2, x1, w2rel, w2root, b2)
    x2 = _tc(_bn_body, jax.ShapeDtypeStruct((N, 128), f32), t2, g2, be2)

    agg3 = _segsum(x2, src2d, dst2d)
    t3 = _lin(agg3, x2, w3rel, w3root, b3)
    x3 = _tc(_bn_body, jax.ShapeDtypeStruct((N, 192), f32), t3, g3, be3)

    so_stack = pl.pallas_call(
        _som_body,
        grid=(N // RB,),
        in_specs=[
            pl.BlockSpec((RB, 64), lambda i: (i, 0)),
            pl.BlockSpec((RB, 128), lambda i: (i, 0)),
            pl.BlockSpec((RB, 192), lambda i: (i, 0)),
            full((64, 128)), full((1, 128)),
            full((128, 128)), full((1, 128)),
            full((192, 128)), full((1, 128)),
        ],
        out_specs=pl.BlockSpec((2, RB, SOW), lambda i: (0, i, 0)),
        out_shape=jax.ShapeDtypeStruct((2, N, SOW), f32),
    )(x1, x2, x3, s1t, s1n, s2t, s2n, s3t, s3n)

    agg_so = _segsum_oc(so_stack.reshape(2 * N, SOW), src2d, dst2d)

    t_oc = pl.pallas_call(
        _oc_body,
        grid=(N // RB,),
        in_specs=[
            pl.BlockSpec((2, RB, SOW), lambda i: (0, i, 0)),
            pl.BlockSpec((2, RB, SOW), lambda i: (0, i, 0)),
            full((100, 64)), full((100, 64)),
            full((100, 64)), full((100, 64)),
            full((100, 64)), full((100, 64)),
            full((1, 192)),
        ],
        out_specs=pl.BlockSpec((RB, 192), lambda i: (i, 0)),
        out_shape=jax.ShapeDtypeStruct((N, 192), f32),
    )(agg_so, so_stack, wr1, wo1, wr2, wo2, wr3, wo3, boc)
    som_out = _tc(_bn_body, jax.ShapeDtypeStruct((N, 192), f32),
                  t_oc, goc, beoc)

    h_conv = jnp.concatenate([x1, x2, x3], axis=1)
    h, gnn_out = pl.pallas_call(
        _pool_body,
        grid=(PNSTEP,),
        in_specs=[
            pl.BlockSpec((PRB, 384), lambda i: (i, 0)),
            pl.BlockSpec((PRB, 192), lambda i: (i, 0)),
            pl.BlockSpec((PRB, 1), lambda i: (i, 0)),
            full((1152, 2)), full((1, 2)), full((576, 2)), full((1, 2)),
        ],
        out_specs=[full((64, 2)), full((64, 2))],
        out_shape=[jax.ShapeDtypeStruct((64, 2), f32),
                   jax.ShapeDtypeStruct((64, 2), f32)],
        scratch_shapes=[
            pltpu.VMEM((64, 384), f32),
            pltpu.VMEM((64, 192), f32),
            pltpu.VMEM((64, 384), f32),
            pltpu.VMEM((64, 192), f32),
            pltpu.VMEM((64, 1), f32),
        ],
    )(h_conv, som_out, batch2d, wgnn, bgnn, wout, bout)
    return (h, h_conv, gnn_out)


# speedup vs baseline: 6.1944x; 1.1077x over previous
import functools

import jax
import jax.numpy as jnp
from jax import lax
from jax.experimental import pallas as pl
from jax.experimental.pallas import tpu as pltpu
from jax.experimental.pallas import tpu_sc as plsc

N = 10000
N_PAD = 10240
E = 320000
NCORES = 2
NSUB = 16
NW = NCORES * NSUB
EPW = E // NW
CHUNK = 80
NCHUNK = EPW // CHUNK
ROWS_PER_TILE = N_PAD // NSUB
SOW = 160


def _seg_pipeline(vals, acc, src2d, dst2d, rowbase, bias,
                  idx_s, idx_d, rows, gsem, ssem, isem, nch):
    nbuf = len(rows)
    nfull = nch // nbuf
    rem = nch % nbuf

    def idx_start(j, b):
        return (pltpu.async_copy(src2d.at[rowbase + j], idx_s[b], isem[b]),
                pltpu.async_copy(dst2d.at[rowbase + j], idx_d[b], isem[b]))

    def gather_start(ids, b):
        for d in ids:
            d.wait()
        if bias is not None:
            for k in range(CHUNK // 16):
                sl = pl.ds(k * 16, 16)
                idx_s[b][sl] = idx_s[b][sl] + bias
        return pltpu.async_copy(vals.at[idx_s[b]], rows[b], gsem[b])

    def group(i, carry):
        ids = []
        for b in range(nbuf):
            @pl.when(i > 0)
            def _(b=b):
                pltpu.make_async_copy(rows[b], acc.at[idx_d[b]],
                                      ssem[b]).wait()
            ids.append(idx_start(i * nbuf + b, b))
        gds = [gather_start(ids[b], b) for b in range(nbuf)]
        for b in range(nbuf):
            gds[b].wait()
            pltpu.async_copy(rows[b], acc.at[idx_d[b]], ssem[b], add=True)
        return carry

    lax.fori_loop(0, nfull, group, 0)
    for b in range(rem):
        if nfull > 0:
            pltpu.make_async_copy(rows[b], acc.at[idx_d[b]], ssem[b]).wait()
        gather_start(idx_start(nfull * nbuf + b, b), b).wait()
        pltpu.async_copy(rows[b], acc.at[idx_d[b]], ssem[b], add=True)
    for b in range(nbuf):
        if nfull > 0 or b < rem:
            pltpu.make_async_copy(rows[b], acc.at[idx_d[b]], ssem[b]).wait()


def _make_segsum_impl(W, split, nbuf):
    NCH = NCHUNK if not split else (E // NSUB) // CHUNK
    mesh = plsc.VectorSubcoreMesh(core_axis_name="c", subcore_axis_name="s",
                                  num_cores=NCORES, num_subcores=NSUB)

    @functools.partial(
        pl.kernel,
        out_type=jax.ShapeDtypeStruct((NCORES, N_PAD, W), jnp.float32),
        mesh=mesh,
        compiler_params=pltpu.CompilerParams(use_tc_tiling_on_sc=False),
        scratch_types=[
            [pltpu.VMEM((CHUNK,), jnp.int32)] * nbuf,
            [pltpu.VMEM((CHUNK,), jnp.int32)] * nbuf,
            [pltpu.VMEM((CHUNK, W), jnp.float32)] * nbuf,
            pltpu.VMEM_SHARED((N_PAD, W), jnp.float32),
            [pltpu.SemaphoreType.DMA] * nbuf,
            [pltpu.SemaphoreType.DMA] * nbuf,
            [pltpu.SemaphoreType.DMA] * nbuf,
        ],
    )
    def seg(vals, src2d, dst2d, zeros, out,
            idx_s, idx_d, rows, acc, gsem, ssem, isem):
        cid = lax.axis_index("c")
        sid = lax.axis_index("s")
        r0 = sid * ROWS_PER_TILE
        pltpu.sync_copy(zeros.at[pl.ds(r0, ROWS_PER_TILE)],
                        acc.at[pl.ds(r0, ROWS_PER_TILE)])
        plsc.subcore_barrier()
        if split:
            rowbase = sid * NCH
            bias = cid * N
        else:
            rowbase = (cid * NSUB + sid) * NCH
            bias = None
        _seg_pipeline(vals, acc, src2d, dst2d, rowbase, bias,
                      idx_s, idx_d, rows, gsem, ssem, isem, NCH)
        plsc.subcore_barrier()
        pltpu.sync_copy(acc.at[pl.ds(r0, ROWS_PER_TILE)],
                        out.at[cid, pl.ds(r0, ROWS_PER_TILE)])

    return seg


_SEGSUM_CACHE = {}


def _segsum(vals, src2d, dst2d):
    W = vals.shape[1]
    if W not in _SEGSUM_CACHE:
        _SEGSUM_CACHE[W] = _make_segsum_impl(W, split=False, nbuf=4)
    zeros = jnp.zeros((N_PAD, W), jnp.float32)
    return _SEGSUM_CACHE[W](vals, src2d, dst2d, zeros)


def _segsum_oc(vals2n, src2d, dst2d):
    if 'oc' not in _SEGSUM_CACHE:
        _SEGSUM_CACHE['oc'] = _make_segsum_impl(SOW, split=True, nbuf=2)
    zeros = jnp.zeros((N_PAD, SOW), jnp.float32)
    return _SEGSUM_CACHE['oc'](vals2n, src2d, dst2d, zeros)


def _leaky(t):
    return jnp.where(t >= 0, t, 0.01 * t)


def _bn(t, g, b):
    mu = jnp.mean(t, axis=0, keepdims=True)
    var = jnp.mean((t - mu) ** 2, axis=0, keepdims=True)
    return (t - mu) / jnp.sqrt(var + 1e-5) * g + b


def _lin_body(aggp, x1, wrel, wroot, b, o):
    agg = aggp[0] + aggp[1]
    h = (jnp.dot(agg, wrel[...], preferred_element_type=jnp.float32)
         + jnp.dot(x1[...], wroot[...], preferred_element_type=jnp.float32)
         + b[...])
    o[...] = _leaky(h)


def _bn_body(t, g, be, o):
    o[...] = _bn(t[...], g[...], be[...])


def _som_body(x1, x2, x3, s1t, s1n, s2t, s2n, s3t, s3n, o_so):
    def dists(xi, wt, wn):
        d2 = (jnp.sum(xi * xi, axis=1, keepdims=True)
              - 2.0 * jnp.dot(xi, wt, preferred_element_type=jnp.float32)
              + wn)
        return jnp.sqrt(jnp.maximum(d2, 1e-12))

    so1 = dists(x1[...], s1t[...], s1n[...])[:, :100]
    so2 = dists(x2[...], s2t[...], s2n[...])[:, :100]
    so3 = dists(x3[...], s3t[...], s3n[...])[:, :100]
    zpad = jnp.zeros((so1.shape[0], 20), jnp.float32)
    o_so[0, :, :] = jnp.concatenate([so1, so2[:, :60]], axis=1)
    o_so[1, :, :] = jnp.concatenate([so2[:, 60:], so3, zpad], axis=1)


def _oc_body(aggso, sost, wr1, wo1, wr2, wo2, wr3, wo3, b, o):
    a0, a1 = aggso[0], aggso[1]
    s0, s1 = sost[0], sost[1]
    agg1, so1 = a0[:, :100], s0[:, :100]
    agg2 = jnp.concatenate([a0[:, 100:160], a1[:, :40]], axis=1)
    so2 = jnp.concatenate([s0[:, 100:160], s1[:, :40]], axis=1)
    agg3, so3 = a1[:, 40:140], s1[:, 40:140]

    def part(agg, so, wr, wo):
        return (jnp.dot(agg, wr[...], preferred_element_type=jnp.float32)
                + jnp.dot(so, wo[...], preferred_element_type=jnp.float32))

    t = jnp.concatenate([part(agg1, so1, wr1, wo1),
                         part(agg2, so2, wr2, wo2),
                         part(agg3, so3, wr3, wo3)], axis=1) + b[...]
    o[...] = _leaky(t)


PRB = 2000
PNSTEP = N // PRB


def _pool_body(hc_ref, som_ref, batch_ref, wgnn, bgnn, wout, bout,
               o_h, o_gnn, s_conv, s_som, mx_c, mx_s, cnt):
    i = pl.program_id(0)

    @pl.when(i == 0)
    def _init():
        s_conv[...] = jnp.zeros((64, 384), jnp.float32)
        s_som[...] = jnp.zeros((64, 192), jnp.float32)
        mx_c[...] = jnp.full((64, 384), -jnp.inf, jnp.float32)
        mx_s[...] = jnp.full((64, 192), -jnp.inf, jnp.float32)
        cnt[...] = jnp.zeros((64, 1), jnp.float32)

    bvec = batch_ref[...]
    hc = hc_ref[...]
    som = som_ref[...]
    oh = (bvec == lax.broadcasted_iota(jnp.int32, (PRB, 64), 1)
          ).astype(jnp.float32)

    def segsum(hm):
        return lax.dot_general(oh, hm, (((0,), (0,)), ((), ())),
                               preferred_element_type=jnp.float32,
                               precision=lax.Precision.HIGHEST)

    s_conv[...] += segsum(hc)
    s_som[...] += segsum(som)
    cnt[...] += lax.dot_general(oh, jnp.ones((PRB, 1), jnp.float32),
                                (((0,), (0,)), ((), ())),
                                preferred_element_type=jnp.float32,
                                precision=lax.Precision.HIGHEST)
    row_iota = lax.broadcasted_iota(jnp.int32, (64, 1), 0)

    def mxstep(gidx, carry):
        mc = jnp.max(jnp.where(bvec == gidx, hc, -jnp.inf),
                     axis=0, keepdims=True)
        ms = jnp.max(jnp.where(bvec == gidx, som, -jnp.inf),
                     axis=0, keepdims=True)
        sel = row_iota == gidx
        mx_c[...] = jnp.where(sel, jnp.maximum(mx_c[...], mc), mx_c[...])
        mx_s[...] = jnp.where(sel, jnp.maximum(mx_s[...], ms), mx_s[...])
        return carry

    lax.fori_loop(0, 64, mxstep, 0)

    @pl.when(i == PNSTEP - 1)
    def _heads():
        cn = jnp.maximum(cnt[...], 1.0)
        sc = s_conv[...]
        ss = s_som[...]
        h_gnn = jnp.concatenate([sc / cn, sc, mx_c[...]], axis=1)
        hp = jnp.concatenate([ss / cn, ss, mx_s[...]], axis=1)

        def logsoftmax(z):
            zm = jnp.max(z, axis=1, keepdims=True)
            return (z - zm) - jnp.log(jnp.sum(jnp.exp(z - zm), axis=1,
                                              keepdims=True))

        o_gnn[...] = logsoftmax(
            jnp.dot(h_gnn, wgnn[...], preferred_element_type=jnp.float32)
            + bgnn[...])
        o_h[...] = logsoftmax(
            jnp.dot(hp, wout[...], preferred_element_type=jnp.float32)
            + bout[...])


def _tc(body, out_shape, *args):
    return pl.pallas_call(body, out_shape=out_shape)(*args)


def kernel(x, edge_index, batch, params):
    p = params
    src = edge_index[0]
    dst = edge_index[1]
    src2d = src.reshape(E // CHUNK, CHUNK)
    dst2d = dst.reshape(E // CHUNK, CHUNK)

    w1rel, w1root = p['conv1_Wrel'].T, p['conv1_Wroot'].T
    b1 = p['conv1_b'].reshape(1, -1)
    g1, be1 = p['norm1_g'].reshape(1, -1), p['norm1_b'].reshape(1, -1)
    w2rel, w2root = p['conv2_Wrel'].T, p['conv2_Wroot'].T
    b2 = p['conv2_b'].reshape(1, -1)
    g2, be2 = p['norm2_g'].reshape(1, -1), p['norm2_b'].reshape(1, -1)
    w3rel, w3root = p['conv3_Wrel'].T, p['conv3_Wroot'].T
    b3 = p['conv3_b'].reshape(1, -1)
    g3, be3 = p['norm3_g'].reshape(1, -1), p['norm3_b'].reshape(1, -1)

    def som_prep(w):
        wt = jnp.pad(w, ((0, 28), (0, 0))).T
        wn = jnp.pad(jnp.sum(w * w, axis=1), (0, 28)).reshape(1, -1)
        return wt, wn

    s1t, s1n = som_prep(p['som1_W'])
    s2t, s2n = som_prep(p['som2_W'])
    s3t, s3n = som_prep(p['som3_W'])

    wr1, wo1 = p['oc1_Wrel'].T, p['oc1_Wroot'].T
    wr2, wo2 = p['oc2_Wrel'].T, p['oc2_Wroot'].T
    wr3, wo3 = p['oc3_Wrel'].T, p['oc3_Wroot'].T
    boc = jnp.concatenate([p['oc1_b'], p['oc2_b'], p['oc3_b']]).reshape(1, -1)
    goc = jnp.concatenate([p['on1_g'], p['on2_g'], p['on3_g']]).reshape(1, -1)
    beoc = jnp.concatenate([p['on1_b'], p['on2_b'], p['on3_b']]).reshape(1, -1)
    wgnn = p['lin_GNN_W'].T
    bgnn = p['lin_GNN_b'].reshape(1, -1)
    wout = p['lin_out_W'].T
    bout = p['lin_out_b'].reshape(1, -1)
    batch2d = batch.reshape(N, 1)

    f32 = jnp.float32
    RB = 2000
    full = lambda shp: pl.BlockSpec(shp, lambda i: (0,) * len(shp))

    def _lin(aggp, xin, wrel, wroot, bb):
        wa, wx, wo = aggp.shape[2], xin.shape[1], wrel.shape[1]
        return pl.pallas_call(
            _lin_body,
            grid=(N // RB,),
            in_specs=[
                pl.BlockSpec((2, RB, wa), lambda i: (0, i, 0)),
                pl.BlockSpec((RB, wx), lambda i: (i, 0)),
                full((wa, wo)), full((wx, wo)), full((1, wo)),
            ],
            out_specs=pl.BlockSpec((RB, wo), lambda i: (i, 0)),
            out_shape=jax.ShapeDtypeStruct((N, wo), f32),
        )(aggp, xin, wrel, wroot, bb)

    agg1 = _segsum(x, src2d, dst2d)
    t1 = _lin(agg1, x, w1rel, w1root, b1)
    x1 = _tc(_bn_body, jax.ShapeDtypeStruct((N, 64), f32), t1, g1, be1)

    agg2 = _segsum(x1, src2d, dst2d)
    t2 = _lin(agg2, x1, w2rel, w2root, b2)
    x2 = _tc(_bn_body, jax.ShapeDtypeStruct((N, 128), f32), t2, g2, be2)

    agg3 = _segsum(x2, src2d, dst2d)
    t3 = _lin(agg3, x2, w3rel, w3root, b3)
    x3 = _tc(_bn_body, jax.ShapeDtypeStruct((N, 192), f32), t3, g3, be3)

    so_stack = pl.pallas_call(
        _som_body,
        grid=(N // RB,),
        in_specs=[
            pl.BlockSpec((RB, 64), lambda i: (i, 0)),
            pl.BlockSpec((RB, 128), lambda i: (i, 0)),
            pl.BlockSpec((RB, 192), lambda i: (i, 0)),
            full((64, 128)), full((1, 128)),
            full((128, 128)), full((1, 128)),
            full((192, 128)), full((1, 128)),
        ],
        out_specs=pl.BlockSpec((2, RB, SOW), lambda i: (0, i, 0)),
        out_shape=jax.ShapeDtypeStruct((2, N, SOW), f32),
    )(x1, x2, x3, s1t, s1n, s2t, s2n, s3t, s3n)

    agg_so = _segsum_oc(so_stack.reshape(2 * N, SOW), src2d, dst2d)

    t_oc = pl.pallas_call(
        _oc_body,
        grid=(N // RB,),
        in_specs=[
            pl.BlockSpec((2, RB, SOW), lambda i: (0, i, 0)),
            pl.BlockSpec((2, RB, SOW), lambda i: (0, i, 0)),
            full((100, 64)), full((100, 64)),
            full((100, 64)), full((100, 64)),
            full((100, 64)), full((100, 64)),
            full((1, 192)),
        ],
        out_specs=pl.BlockSpec((RB, 192), lambda i: (i, 0)),
        out_shape=jax.ShapeDtypeStruct((N, 192), f32),
    )(agg_so, so_stack, wr1, wo1, wr2, wo2, wr3, wo3, boc)
    som_out = _tc(_bn_body, jax.ShapeDtypeStruct((N, 192), f32),
                  t_oc, goc, beoc)

    h_conv = jnp.concatenate([x1, x2, x3], axis=1)
    h, gnn_out = pl.pallas_call(
        _pool_body,
        grid=(PNSTEP,),
        in_specs=[
            pl.BlockSpec((PRB, 384), lambda i: (i, 0)),
            pl.BlockSpec((PRB, 192), lambda i: (i, 0)),
            pl.BlockSpec((PRB, 1), lambda i: (i, 0)),
            full((1152, 2)), full((1, 2)), full((576, 2)), full((1, 2)),
        ],
        out_specs=[full((64, 2)), full((64, 2))],
        out_shape=[jax.ShapeDtypeStruct((64, 2), f32),
                   jax.ShapeDtypeStruct((64, 2), f32)],
        scratch_shapes=[
            pltpu.VMEM((64, 384), f32),
            pltpu.VMEM((64, 192), f32),
            pltpu.VMEM((64, 384), f32),
            pltpu.VMEM((64, 192), f32),
            pltpu.VMEM((64, 1), f32),
        ],
    )(h_conv, som_out, batch2d, wgnn, bgnn, wout, bout)
    return (h, h_conv, gnn_out)
